# baseline (speedup 1.0000x reference)
import functools

import jax
import jax.numpy as jnp
from jax import lax
from jax.experimental import pallas as pl
from jax.experimental.pallas import tpu as pltpu
from jax.experimental.pallas import tpu_sc as plsc

_N = 1024
_M = 512
_D = 256

_NC = 2
_NS = 16
_L = 16
_NW = _NC * _NS
_RPW = _N // _NW


def _tc_topk_kernel(q_ref, k_ref, qn_ref, idx1_ref, idx2_ref):
    k = k_ref[...]

    ones = jnp.ones((1, _D), jnp.float32)
    ksq = jax.lax.dot_general(
        ones, k * k, (((1,), (1,)), ((), ())),
        preferred_element_type=jnp.float32,
        precision=jax.lax.Precision.HIGHEST,
    )

    q = q_ref[...]

    norm = jnp.sqrt(jnp.sum(q * q, axis=1, keepdims=True))
    qn = q / jnp.maximum(norm, 1e-12)
    qn_ref[...] = qn

    qk = jax.lax.dot_general(
        qn, k, (((1,), (1,)), ((), ())), preferred_element_type=jnp.float32,
        precision=jax.lax.Precision.HIGHEST,
    )
    dist = ksq - 2.0 * qk

    col = jax.lax.broadcasted_iota(jnp.int32, (_N, _M), 1)
    dmin1 = jnp.min(dist, axis=1, keepdims=True)
    idx1 = jnp.min(jnp.where(dist <= dmin1, col, _M), axis=1, keepdims=True)
    dist2 = jnp.where(col == idx1, jnp.inf, dist)
    dmin2 = jnp.min(dist2, axis=1, keepdims=True)
    idx2 = jnp.min(jnp.where(dist2 <= dmin2, col, _M), axis=1, keepdims=True)
    idx1_ref[...] = idx1
    idx2_ref[...] = idx2


def _lane_allsum(v, lanes):
    dnums = lax.GatherDimensionNumbers(
        offset_dims=(), collapsed_slice_dims=(0,), start_index_map=(0,))
    for s in (8, 4, 2, 1):
        v = v + lax.gather(
            v, (lanes ^ s)[:, None], dnums, slice_sizes=(1,),
            mode=lax.GatherScatterMode.PROMISE_IN_BOUNDS)
    return v


def _sc_rescore_kernel(keys_hbm, qn_hbm, idx1_hbm, idx2_hbm, out_hbm,
                       idx1_v, idx2_v, qn_v, g1_v, g2_v, out_v, sem):
    wid = lax.axis_index("s") * _NC + lax.axis_index("c")
    base = wid * _RPW

    pltpu.sync_copy(idx1_hbm.at[pl.ds(base, _RPW)], idx1_v)
    pltpu.sync_copy(idx2_hbm.at[pl.ds(base, _RPW)], idx2_v)
    pltpu.sync_copy(qn_hbm.at[pl.ds(base, _RPW)], qn_v)

    pltpu.async_copy(keys_hbm.at[idx1_v], g1_v, sem).wait()
    pltpu.async_copy(keys_hbm.at[idx2_v], g2_v, sem).wait()

    lanes = lax.broadcasted_iota(jnp.int32, (_L,), 0)

    for g in range(_RPW // _L):
        def row_body(r, carry, g=g):
            d1v, d2v, l1v, l2v = carry
            rr = g * _L + r
            d1 = jnp.zeros((_L,), jnp.float32)
            d2 = jnp.zeros((_L,), jnp.float32)
            l1 = jnp.zeros((_L,), jnp.float32)
            l2 = jnp.zeros((_L,), jnp.float32)
            for j in range(_D // _L):
                q = qn_v[rr, pl.ds(j * _L, _L)]
                a = q - g1_v[rr, pl.ds(j * _L, _L)]
                b = q - g2_v[rr, pl.ds(j * _L, _L)]
                a2 = a * a
                b2 = b * b
                d1 = d1 + a2
                d2 = d2 + b2
                l1 = l1 + a2 * a2
                l2 = l2 + b2 * b2
            hit = lanes == r
            d1v = jnp.where(hit, _lane_allsum(d1, lanes), d1v)
            d2v = jnp.where(hit, _lane_allsum(d2, lanes), d2v)
            l1v = jnp.where(hit, _lane_allsum(l1, lanes), l1v)
            l2v = jnp.where(hit, _lane_allsum(l2, lanes), l2v)
            return (d1v, d2v, l1v, l2v)

        z = jnp.zeros((_L,), jnp.float32)
        d1v, d2v, l1v, l2v = lax.fori_loop(0, _L, row_body, (z, z, z, z))

        iv1 = idx1_v[pl.ds(g * _L, _L)]
        iv2 = idx2_v[pl.ds(g * _L, _L)]
        take2 = (d2v < d1v) | ((d2v == d1v) & (iv2 < iv1))
        out_v[pl.ds(g * _L, _L)] = jnp.where(take2, l2v, l1v)

    pltpu.sync_copy(out_v, out_hbm.at[pl.ds(base, _RPW)])


@functools.partial(
    pl.kernel,
    out_type=jax.ShapeDtypeStruct((_N,), jnp.float32),
    mesh=plsc.VectorSubcoreMesh(core_axis_name="c", subcore_axis_name="s"),
    scratch_types=[
        pltpu.VMEM((_RPW,), jnp.int32),
        pltpu.VMEM((_RPW,), jnp.int32),
        pltpu.VMEM((_RPW, _D), jnp.float32),
        pltpu.VMEM((_RPW, _D), jnp.float32),
        pltpu.VMEM((_RPW, _D), jnp.float32),
        pltpu.VMEM((_RPW,), jnp.float32),
        pltpu.SemaphoreType.DMA,
    ],
)
def _sc_rescore(keys_hbm, qn_hbm, idx1_hbm, idx2_hbm, out_hbm,
                idx1_v, idx2_v, qn_v, g1_v, g2_v, out_v, sem):
    _sc_rescore_kernel(keys_hbm, qn_hbm, idx1_hbm, idx2_hbm, out_hbm,
                       idx1_v, idx2_v, qn_v, g1_v, g2_v, out_v, sem)


def kernel(query, keys, train):
    q = query[0]
    b, c, h, w = q.shape
    qr = jnp.transpose(q, (0, 2, 3, 1)).reshape(b * h * w, c)
    qn, idx1, idx2 = pl.pallas_call(
        _tc_topk_kernel,
        out_shape=(
            jax.ShapeDtypeStruct((_N, _D), jnp.float32),
            jax.ShapeDtypeStruct((_N, 1), jnp.int32),
            jax.ShapeDtypeStruct((_N, 1), jnp.int32),
        ),
    )(qr, keys[0])
    loss = _sc_rescore(keys[0], qn, idx1.reshape(_N), idx2.reshape(_N))
    return loss.reshape(b, h, w, 1)


# speedup vs baseline: 1.3154x; 1.3154x over previous
import functools

import jax
import jax.numpy as jnp
from jax import lax
from jax.experimental import pallas as pl
from jax.experimental.pallas import tpu as pltpu
from jax.experimental.pallas import tpu_sc as plsc

_N = 1024
_M = 512
_D = 256

_NC = 2
_NS = 16
_L = 16
_NW = _NC * _NS
_RPW = _N // _NW


def _tc_topk_kernel(q_ref, k_ref, qn_ref, idx1_ref, idx2_ref):
    k = k_ref[...]

    ones = jnp.ones((1, _D), jnp.float32)
    ksq = jax.lax.dot_general(
        ones, k * k, (((1,), (1,)), ((), ())),
        preferred_element_type=jnp.float32,
        precision=jax.lax.Precision.HIGHEST,
    )

    q = q_ref[...]

    norm = jnp.sqrt(jnp.sum(q * q, axis=1, keepdims=True))
    qn = q / jnp.maximum(norm, 1e-12)
    qn_ref[...] = qn

    qk = jax.lax.dot_general(
        qn, k, (((1,), (1,)), ((), ())), preferred_element_type=jnp.float32,
        precision=jax.lax.Precision.HIGHEST,
    )
    dist = ksq - 2.0 * qk

    col = jax.lax.broadcasted_iota(jnp.int32, (_N, _M), 1)
    dmin1 = jnp.min(dist, axis=1, keepdims=True)
    idx1 = jnp.min(jnp.where(dist <= dmin1, col, _M), axis=1, keepdims=True)
    dist2 = jnp.where(col == idx1, jnp.inf, dist)
    dmin2 = jnp.min(dist2, axis=1, keepdims=True)
    idx2 = jnp.min(jnp.where(dist2 <= dmin2, col, _M), axis=1, keepdims=True)
    idx1_ref[...] = idx1
    idx2_ref[...] = idx2


def _lane_allsum(v, lanes):
    dnums = lax.GatherDimensionNumbers(
        offset_dims=(), collapsed_slice_dims=(0,), start_index_map=(0,))
    for s in (8, 4, 2, 1):
        v = v + lax.gather(
            v, (lanes ^ s)[:, None], dnums, slice_sizes=(1,),
            mode=lax.GatherScatterMode.PROMISE_IN_BOUNDS)
    return v


def _sc_rescore_kernel(keys_hbm, qn_hbm, idx1_hbm, idx2_hbm, out_hbm,
                       idx1_v, idx2_v, qn_v, g1_v, g2_v, out_v, sem):
    sid = lax.axis_index("s")
    wid = sid * _NC + lax.axis_index("c")
    base = wid * _RPW

    pltpu.sync_copy(idx1_hbm.at[pl.ds(base, _RPW)], idx1_v)
    pltpu.sync_copy(idx2_hbm.at[pl.ds(base, _RPW)], idx2_v)
    pltpu.sync_copy(qn_hbm.at[pl.ds(base, _RPW)], qn_v)

    c1 = pltpu.async_copy(keys_hbm.at[idx1_v], g1_v, sem)
    c2 = pltpu.async_copy(keys_hbm.at[idx2_v], g2_v, sem)
    c1.wait()
    c2.wait()

    lanes = lax.broadcasted_iota(jnp.int32, (_L,), 0)

    for g in range(_RPW // _L):
        def row_body(r, carry, g=g):
            d1v, d2v, l1v, l2v = carry
            rr = g * _L + r
            d1 = jnp.zeros((_L,), jnp.float32)
            d2 = jnp.zeros((_L,), jnp.float32)
            l1 = jnp.zeros((_L,), jnp.float32)
            l2 = jnp.zeros((_L,), jnp.float32)
            for j in range(_D // _L):
                s, o = divmod(j * _L, 128)
                q = qn_v[rr, pl.ds(j * _L, _L)]
                a = q - g1_v[rr, s, pl.ds(o, _L)]
                b = q - g2_v[rr, s, pl.ds(o, _L)]
                a2 = a * a
                b2 = b * b
                d1 = d1 + a2
                d2 = d2 + b2
                l1 = l1 + a2 * a2
                l2 = l2 + b2 * b2
            hit = lanes == r
            d1v = jnp.where(hit, _lane_allsum(d1, lanes), d1v)
            d2v = jnp.where(hit, _lane_allsum(d2, lanes), d2v)
            l1v = jnp.where(hit, _lane_allsum(l1, lanes), l1v)
            l2v = jnp.where(hit, _lane_allsum(l2, lanes), l2v)
            return (d1v, d2v, l1v, l2v)

        z = jnp.zeros((_L,), jnp.float32)
        d1v, d2v, l1v, l2v = lax.fori_loop(0, _L, row_body, (z, z, z, z))

        iv1 = idx1_v[pl.ds(g * _L, _L)]
        iv2 = idx2_v[pl.ds(g * _L, _L)]
        take2 = (d2v < d1v) | ((d2v == d1v) & (iv2 < iv1))
        out_v[pl.ds(g * _L, _L)] = jnp.where(take2, l2v, l1v)

    pltpu.sync_copy(out_v, out_hbm.at[pl.ds(base, _RPW)])


@functools.partial(
    pl.kernel,
    out_type=jax.ShapeDtypeStruct((_N,), jnp.float32),
    mesh=plsc.VectorSubcoreMesh(core_axis_name="c", subcore_axis_name="s"),
    scratch_types=[
        pltpu.VMEM((_RPW,), jnp.int32),
        pltpu.VMEM((_RPW,), jnp.int32),
        pltpu.VMEM((_RPW, _D), jnp.float32),
        pltpu.VMEM((_RPW, _D // 128, 128), jnp.float32),
        pltpu.VMEM((_RPW, _D // 128, 128), jnp.float32),
        pltpu.VMEM((_RPW,), jnp.float32),
        pltpu.SemaphoreType.DMA,
    ],
)
def _sc_rescore(keys_hbm, qn_hbm, idx1_hbm, idx2_hbm, out_hbm,
                idx1_v, idx2_v, qn_v, g1_v, g2_v, out_v, sem):
    _sc_rescore_kernel(keys_hbm, qn_hbm, idx1_hbm, idx2_hbm, out_hbm,
                       idx1_v, idx2_v, qn_v, g1_v, g2_v, out_v, sem)


def kernel(query, keys, train):
    q = query[0]
    b, c, h, w = q.shape
    qr = jnp.transpose(q, (0, 2, 3, 1)).reshape(b * h * w, c)
    qn, idx1, idx2 = pl.pallas_call(
        _tc_topk_kernel,
        out_shape=(
            jax.ShapeDtypeStruct((_N, _D), jnp.float32),
            jax.ShapeDtypeStruct((_N, 1), jnp.int32),
            jax.ShapeDtypeStruct((_N, 1), jnp.int32),
        ),
    )(qr, keys[0])
    loss = _sc_rescore(keys[0].reshape(_M, _D // 128, 128), qn,
                       idx1.reshape(_N), idx2.reshape(_N))
    return loss.reshape(b, h, w, 1)


# speedup vs baseline: 9.7632x; 7.4220x over previous
import jax
import jax.numpy as jnp
from jax.experimental import pallas as pl
from jax.experimental.pallas import tpu as pltpu

_N = 1024
_M = 512
_D = 256
_BN = 1024


def _sel(onehot, part):
    return jax.lax.dot_general(
        onehot, part, (((1,), (0,)), ((), ())),
        preferred_element_type=jnp.float32,
    )


def _bf16_exact_part(x):
    u = jax.lax.bitcast_convert_type(x, jnp.uint32)
    return jax.lax.bitcast_convert_type(u & jnp.uint32(0xFFFF0000), jnp.float32)


def _hotmap_kernel(q_ref, k_ref, out_ref, ksq_ref, khi_ref, kmid_ref, klo_ref):
    k = k_ref[...]

    @pl.when(pl.program_id(0) == 0)
    def _init():
        ones = jnp.ones((1, _D), jnp.float32)
        ksq_ref[...] = jax.lax.dot_general(
            ones, k * k, (((1,), (1,)), ((), ())),
            preferred_element_type=jnp.float32,
            precision=jax.lax.Precision.HIGHEST,
        )
        khi = _bf16_exact_part(k)
        r1 = k - khi
        kmid = _bf16_exact_part(r1)
        khi_ref[...] = khi
        kmid_ref[...] = kmid
        klo_ref[...] = r1 - kmid

    q = q_ref[...]

    norm = jnp.sqrt(jnp.sum(q * q, axis=1, keepdims=True))
    qn = q / jnp.maximum(norm, 1e-12)

    qk = jax.lax.dot_general(
        qn, k, (((1,), (1,)), ((), ())), preferred_element_type=jnp.float32,
        precision=jax.lax.Precision.HIGHEST,
    )
    dist = ksq_ref[...] - 2.0 * qk

    col = jax.lax.broadcasted_iota(jnp.int32, (_BN, _M), 1)
    dmin1 = jnp.min(dist, axis=1, keepdims=True)
    idx1 = jnp.min(jnp.where(dist <= dmin1, col, _M), axis=1, keepdims=True)
    hit1 = col == idx1
    dist2 = jnp.where(hit1, jnp.inf, dist)
    dmin2 = jnp.min(dist2, axis=1, keepdims=True)
    idx2 = jnp.min(jnp.where(dist2 <= dmin2, col, _M), axis=1, keepdims=True)
    hit2 = col == idx2

    khi = khi_ref[...]
    kmid = kmid_ref[...]
    klo = klo_ref[...]
    oh1 = hit1.astype(jnp.float32)
    oh2 = hit2.astype(jnp.float32)
    g1 = _sel(oh1, khi) + _sel(oh1, kmid) + _sel(oh1, klo)
    g2 = _sel(oh2, khi) + _sel(oh2, kmid) + _sel(oh2, klo)

    e1 = qn - g1
    e2 = qn - g2
    d1 = jnp.sum(e1 * e1, axis=1, keepdims=True)
    d2 = jnp.sum(e2 * e2, axis=1, keepdims=True)
    take2 = (d2 < d1) | ((d2 == d1) & (idx2 < idx1))
    diff = jnp.where(take2, e2, e1)
    d2q = diff * diff
    out_ref[...] = jnp.sum(d2q * d2q, axis=1, keepdims=True)


def kernel(query, keys, train):
    q = query[0]
    b, c, h, w = q.shape
    qr = jnp.transpose(q, (0, 2, 3, 1)).reshape(b * h * w, c)
    loss = pl.pallas_call(
        _hotmap_kernel,
        grid=(_N // _BN,),
        in_specs=[
            pl.BlockSpec((_BN, _D), lambda i: (i, 0)),
            pl.BlockSpec((_M, _D), lambda i: (0, 0)),
        ],
        out_specs=pl.BlockSpec((_BN, 1), lambda i: (i, 0)),
        out_shape=jax.ShapeDtypeStruct((_N, 1), jnp.float32),
        scratch_shapes=[
            pltpu.VMEM((1, _M), jnp.float32),
            pltpu.VMEM((_M, _D), jnp.float32),
            pltpu.VMEM((_M, _D), jnp.float32),
            pltpu.VMEM((_M, _D), jnp.float32),
        ],
    )(qr, keys[0])
    return loss.reshape(b, h, w, 1)
